# baseline (speedup 1.0000x reference)
import functools

import jax
import jax.numpy as jnp
from jax import lax
from jax.experimental import pallas as pl
from jax.experimental.pallas import tpu as pltpu
from jax.experimental.pallas import tpu_sc as plsc

_B, _N = 16384, 200
_NW = 32
_BR = 64
_BLK = _BR * _N
_HB = _BR * 128
_L = 16

_CHUNKS = [16 * k for k in range(12)] + [184]


def _sc_gather(
    x_hbm,
    lab_hbm,
    oute_hbm,
    outo_hbm,
    sum_hbm,
    x_v,
    l_v,
    ge_v,
    go_v,
    acc_v,
    sx0,
    sx1,
    sl0,
    sl1,
    sg0,
    sg1,
    *,
    rows,
):
    wid = lax.axis_index("s") * 2 + lax.axis_index("c")
    rows_per_w = rows // _NW
    nblk = rows_per_w // _BR
    row0 = wid * rows_per_w
    hbase0 = wid * rows_per_w * 128
    sxs, sls, sgs = (sx0, sx1), (sl0, sl1), (sg0, sg1)
    tail_mask = lax.iota(jnp.int32, _L) >= 8

    def start_in(blk):
        off = row0 + blk * _BR
        buf = blk % 2
        cx = pltpu.async_copy(
            x_hbm.at[pl.ds(off, _BR)],
            x_v.at[pl.ds(buf * _BR, _BR)],
            sxs[buf],
        )
        cl = pltpu.async_copy(
            lab_hbm.at[pl.ds(off, _BR)],
            l_v.at[pl.ds(buf * _BR, _BR)],
            sls[buf],
        )
        return cx, cl

    pend = start_in(0)
    gout = [None, None]
    acc = jnp.zeros((_L,), jnp.float32)
    for blk in range(nblk):
        buf = blk % 2
        cx, cl = pend
        cx.wait()
        cl.wait()
        if blk + 1 < nblk:
            pend = start_in(blk + 1)
        if gout[buf] is not None:
            for c in gout[buf]:
                c.wait()

        @plsc.parallel_loop(0, _BR, unroll=4)
        def gbody(row):
            rowi = buf * _BR + row
            rvec = jnp.full((_L,), rowi, jnp.int32)
            for col in _CHUNKS:
                lab = l_v[rowi, pl.ds(col, _L)]
                g = plsc.load_gather(x_v, [rvec, lab])
                if col < 128:
                    ge_v[pl.ds(buf * _HB + row * 128 + col, _L)] = g
                else:
                    go_v[pl.ds(buf * _HB + row * 128 + col - 128, _L)] = g

        @plsc.parallel_loop(0, _BR, unroll=4, carry=acc)
        def sbody(row, sacc):
            rowi = buf * _BR + row
            for col in range(0, 192, 16):
                sacc = sacc + x_v[rowi, pl.ds(col, _L)]
            tail = x_v[rowi, pl.ds(184, _L)]
            sacc = sacc + jnp.where(tail_mask, tail, 0.0)
            return sacc

        acc = sbody
        hoff = hbase0 + blk * _HB
        ce = pltpu.async_copy(
            ge_v.at[pl.ds(buf * _HB, _HB)],
            oute_hbm.at[pl.ds(hoff, _HB)],
            sgs[buf],
        )
        co = pltpu.async_copy(
            go_v.at[pl.ds(buf * _HB, _HB)],
            outo_hbm.at[pl.ds(hoff, _HB)],
            sgs[buf],
        )
        gout[buf] = (ce, co)
    for pair in gout:
        if pair is not None:
            for c in pair:
                c.wait()
    acc_v[...] = acc
    pltpu.sync_copy(acc_v, sum_hbm.at[pl.ds(wid * _L, _L)])


def _tc_body(ge_ref, go_ref, u_ref, s_ref, out_ref, *, n, nblocks, denom):
    i = pl.program_id(0)
    ge = ge_ref[...]
    go = go_ref[...]
    r = ge.shape[0]

    vo = lax.broadcasted_iota(jnp.int32, (r, 128), 1) < (n - 128)
    m = jnp.maximum(
        jnp.max(ge, axis=1, keepdims=True),
        jnp.max(jnp.where(vo, go, -jnp.inf), axis=1, keepdims=True),
    )
    ee = jnp.exp(ge - m)
    eo = jnp.where(vo, jnp.exp(go - m), 0.0)

    u = u_ref[...]
    dims = (((1,), (0,)), ((), ()))

    def psum(e):
        e_hi = e.astype(jnp.bfloat16)
        e_lo = (e - e_hi.astype(jnp.float32)).astype(jnp.bfloat16)
        return lax.dot_general(
            e_hi, u, dims, preferred_element_type=jnp.float32
        ) + lax.dot_general(e_lo, u, dims, preferred_element_type=jnp.float32)

    ce = psum(ee)
    co = psum(eo) + ce[:, 127:128]

    scores_sum = (
        jnp.sum(jnp.log(ce))
        + jnp.sum(jnp.where(vo, jnp.log(co), 0.0))
        + n * jnp.sum(m)
    )

    @pl.when(i == 0)
    def _():
        out_ref[0, 0] = 0.0

    out_ref[0, 0] += scores_sum

    @pl.when(i == nblocks - 1)
    def _():
        out_ref[0, 0] = out_ref[0, 0] - jnp.sum(s_ref[...])


_C = 1


def kernel(outputs, labels):
    b, n = outputs.shape
    rows_c = b // _C

    sc = pl.kernel(
        functools.partial(_sc_gather, rows=rows_c),
        mesh=plsc.VectorSubcoreMesh(core_axis_name="c", subcore_axis_name="s"),
        out_type=(
            jax.ShapeDtypeStruct((rows_c * 128,), jnp.float32),
            jax.ShapeDtypeStruct((rows_c * 128,), jnp.float32),
            jax.ShapeDtypeStruct((_NW * _L,), jnp.float32),
        ),
        scratch_types=[
            pltpu.VMEM((2 * _BR, _N), jnp.float32),
            pltpu.VMEM((2 * _BR, _N), jnp.int32),
            pltpu.VMEM((2 * _HB,), jnp.float32),
            pltpu.VMEM((2 * _HB,), jnp.float32),
            pltpu.VMEM((_L,), jnp.float32),
            pltpu.SemaphoreType.DMA,
            pltpu.SemaphoreType.DMA,
            pltpu.SemaphoreType.DMA,
            pltpu.SemaphoreType.DMA,
            pltpu.SemaphoreType.DMA,
            pltpu.SemaphoreType.DMA,
        ],
        compiler_params=pltpu.CompilerParams(
            needs_layout_passes=False, use_tc_tiling_on_sc=True
        ),
    )

    r = 4096
    nblocks = rows_c // r
    body = functools.partial(_tc_body, n=n, nblocks=nblocks, denom=None)
    u = (
        lax.broadcasted_iota(jnp.int32, (128, 128), 0)
        <= lax.broadcasted_iota(jnp.int32, (128, 128), 1)
    ).astype(jnp.bfloat16)
    tc = pl.pallas_call(
        body,
        grid=(nblocks,),
        in_specs=[
            pl.BlockSpec((r, 128), lambda i: (i, 0)),
            pl.BlockSpec((r, 128), lambda i: (i, 0)),
            pl.BlockSpec((128, 128), lambda i: (0, 0)),
            pl.BlockSpec((4, 128), lambda i: (0, 0)),
        ],
        out_specs=pl.BlockSpec(
            (1, 1), lambda i: (0, 0), memory_space=pltpu.SMEM
        ),
        out_shape=jax.ShapeDtypeStruct((1, 1), jnp.float32),
    )

    labels = labels.astype(jnp.int32)
    total = None
    for c in range(_C):
        sl = slice(c * rows_c, (c + 1) * rows_c)
        ge, go, sums = sc(outputs[sl], labels[sl])
        p = tc(
            ge.reshape(rows_c, 128),
            go.reshape(rows_c, 128),
            u,
            sums.reshape(4, 128),
        )[0, 0]
        total = p if total is None else total + p
    return total * (1.0 / (b * n))


# speedup vs baseline: 1.0155x; 1.0155x over previous
import functools

import jax
import jax.numpy as jnp
from jax import lax
from jax.experimental import pallas as pl
from jax.experimental.pallas import tpu as pltpu
from jax.experimental.pallas import tpu_sc as plsc

_B, _N = 16384, 200
_NW = 32
_BR = 64
_BLK = _BR * _N
_HB = _BR * 128
_L = 16

_CHUNKS = [16 * k for k in range(12)] + [184]


def _sc_gather(
    x_hbm,
    lab_hbm,
    oute_hbm,
    outo_hbm,
    sum_hbm,
    x_v,
    l_v,
    ge_v,
    go_v,
    acc_v,
    sx0,
    sx1,
    sl0,
    sl1,
    sg0,
    sg1,
    *,
    rows,
):
    wid = lax.axis_index("s") * 2 + lax.axis_index("c")
    rows_per_w = rows // _NW
    nblk = rows_per_w // _BR
    row0 = wid * rows_per_w
    hbase0 = wid * rows_per_w * 128
    sxs, sls, sgs = (sx0, sx1), (sl0, sl1), (sg0, sg1)
    tail_mask = lax.iota(jnp.int32, _L) >= 8

    def start_in(blk):
        off = row0 + blk * _BR
        buf = blk % 2
        cx = pltpu.async_copy(
            x_hbm.at[pl.ds(off, _BR)],
            x_v.at[pl.ds(buf * _BR, _BR)],
            sxs[buf],
        )
        cl = pltpu.async_copy(
            lab_hbm.at[pl.ds(off, _BR)],
            l_v.at[pl.ds(buf * _BR, _BR)],
            sls[buf],
        )
        return cx, cl

    pend = start_in(0)
    gout = [None, None]
    acc = jnp.zeros((_L,), jnp.float32)
    for blk in range(nblk):
        buf = blk % 2
        cx, cl = pend
        cx.wait()
        cl.wait()
        if blk + 1 < nblk:
            pend = start_in(blk + 1)
        if gout[buf] is not None:
            for c in gout[buf]:
                c.wait()

        @plsc.parallel_loop(0, _BR, unroll=4, carry=acc)
        def body(row, sacc):
            rowi = buf * _BR + row
            rvec = jnp.full((_L,), rowi, jnp.int32)
            for col in _CHUNKS:
                lab = l_v[rowi, pl.ds(col, _L)]
                g = plsc.load_gather(x_v, [rvec, lab])
                if col < 128:
                    ge_v[pl.ds(buf * _HB + row * 128 + col, _L)] = g
                else:
                    go_v[pl.ds(buf * _HB + row * 128 + col - 128, _L)] = g
            for col in range(0, 192, 16):
                sacc = sacc + x_v[rowi, pl.ds(col, _L)]
            tail = x_v[rowi, pl.ds(184, _L)]
            sacc = sacc + jnp.where(tail_mask, tail, 0.0)
            return sacc

        acc = body
        hoff = hbase0 + blk * _HB
        ce = pltpu.async_copy(
            ge_v.at[pl.ds(buf * _HB, _HB)],
            oute_hbm.at[pl.ds(hoff, _HB)],
            sgs[buf],
        )
        co = pltpu.async_copy(
            go_v.at[pl.ds(buf * _HB, _HB)],
            outo_hbm.at[pl.ds(hoff, _HB)],
            sgs[buf],
        )
        gout[buf] = (ce, co)
    for pair in gout:
        if pair is not None:
            for c in pair:
                c.wait()
    acc_v[...] = acc
    pltpu.sync_copy(acc_v, sum_hbm.at[pl.ds(wid * _L, _L)])


def _tc_body(ge_ref, go_ref, u_ref, s_ref, out_ref, *, n, nblocks, denom):
    i = pl.program_id(0)
    ge = ge_ref[...]
    go = go_ref[...]
    r = ge.shape[0]

    vo = lax.broadcasted_iota(jnp.int32, (r, 128), 1) < (n - 128)
    m = jnp.maximum(
        jnp.max(ge, axis=1, keepdims=True),
        jnp.max(jnp.where(vo, go, -jnp.inf), axis=1, keepdims=True),
    )
    ee = jnp.exp(ge - m)
    eo = jnp.where(vo, jnp.exp(go - m), 0.0)

    u = u_ref[...]
    dims = (((1,), (0,)), ((), ()))

    def psum(e):
        e_hi = e.astype(jnp.bfloat16)
        e_lo = (e - e_hi.astype(jnp.float32)).astype(jnp.bfloat16)
        return lax.dot_general(
            e_hi, u, dims, preferred_element_type=jnp.float32
        ) + lax.dot_general(e_lo, u, dims, preferred_element_type=jnp.float32)

    ce = psum(ee)
    co = psum(eo) + ce[:, 127:128]

    scores_sum = (
        jnp.sum(jnp.log(ce))
        + jnp.sum(jnp.where(vo, jnp.log(co), 0.0))
        + n * jnp.sum(m)
    )

    @pl.when(i == 0)
    def _():
        out_ref[0, 0] = 0.0

    out_ref[0, 0] += scores_sum

    @pl.when(i == nblocks - 1)
    def _():
        out_ref[0, 0] = out_ref[0, 0] - jnp.sum(s_ref[...])


_C = 1


def kernel(outputs, labels):
    b, n = outputs.shape
    rows_c = b // _C

    sc = pl.kernel(
        functools.partial(_sc_gather, rows=rows_c),
        mesh=plsc.VectorSubcoreMesh(core_axis_name="c", subcore_axis_name="s"),
        out_type=(
            jax.ShapeDtypeStruct((rows_c * 128,), jnp.float32),
            jax.ShapeDtypeStruct((rows_c * 128,), jnp.float32),
            jax.ShapeDtypeStruct((_NW * _L,), jnp.float32),
        ),
        scratch_types=[
            pltpu.VMEM((2 * _BR, _N), jnp.float32),
            pltpu.VMEM((2 * _BR, _N), jnp.int32),
            pltpu.VMEM((2 * _HB,), jnp.float32),
            pltpu.VMEM((2 * _HB,), jnp.float32),
            pltpu.VMEM((_L,), jnp.float32),
            pltpu.SemaphoreType.DMA,
            pltpu.SemaphoreType.DMA,
            pltpu.SemaphoreType.DMA,
            pltpu.SemaphoreType.DMA,
            pltpu.SemaphoreType.DMA,
            pltpu.SemaphoreType.DMA,
        ],
        compiler_params=pltpu.CompilerParams(
            needs_layout_passes=False, use_tc_tiling_on_sc=True
        ),
    )

    r = 4096
    nblocks = rows_c // r
    body = functools.partial(_tc_body, n=n, nblocks=nblocks, denom=None)
    u = (
        lax.broadcasted_iota(jnp.int32, (128, 128), 0)
        <= lax.broadcasted_iota(jnp.int32, (128, 128), 1)
    ).astype(jnp.bfloat16)
    tc = pl.pallas_call(
        body,
        grid=(nblocks,),
        in_specs=[
            pl.BlockSpec((r, 128), lambda i: (i, 0)),
            pl.BlockSpec((r, 128), lambda i: (i, 0)),
            pl.BlockSpec((128, 128), lambda i: (0, 0)),
            pl.BlockSpec((4, 128), lambda i: (0, 0)),
        ],
        out_specs=pl.BlockSpec(
            (1, 1), lambda i: (0, 0), memory_space=pltpu.SMEM
        ),
        out_shape=jax.ShapeDtypeStruct((1, 1), jnp.float32),
    )

    labels = labels.astype(jnp.int32)
    total = None
    for c in range(_C):
        sl = slice(c * rows_c, (c + 1) * rows_c)
        ge, go, sums = sc(outputs[sl], labels[sl])
        p = tc(
            ge.reshape(rows_c, 128),
            go.reshape(rows_c, 128),
            u,
            sums.reshape(4, 128),
        )[0, 0]
        total = p if total is None else total + p
    return total * (1.0 / (b * n))
